# baseline (speedup 1.0000x reference)
import functools

import jax
import jax.numpy as jnp
from jax import lax
from jax.experimental import pallas as pl
from jax.experimental.pallas import tpu as pltpu
from jax.experimental.pallas import tpu_sc as plsc

EPS_K = 0.1
L = 16
FILL_UNROLL = 10


def kernel(pred, target):
    n, c = pred.shape
    info = plsc.get_sparse_core_info()
    nc, ns = info.num_cores, info.num_subcores
    nw = nc * ns
    rows_per_w = n // nw
    smooth = EPS_K / (c - 1)
    hot = 1.0 - EPS_K

    mesh = plsc.VectorSubcoreMesh(core_axis_name="c", subcore_axis_name="s")

    @functools.partial(
        pl.kernel,
        out_type=jax.ShapeDtypeStruct((n * c,), jnp.float32),
        mesh=mesh,
        scratch_types=[
            pltpu.VMEM((c,), jnp.float32),
            pltpu.VMEM((rows_per_w,), jnp.int32),
            pltpu.VMEM((rows_per_w,), jnp.int32),
            pltpu.VMEM((rows_per_w,), jnp.float32),
            pltpu.SemaphoreType.DMA,
            pltpu.SemaphoreType.DMA,
        ],
    )
    def sc_kernel(tgt_hbm, out_hbm, row_v, tgt_v, eidx_v, hot_v, sem_fill, sem_hot):
        wid = lax.axis_index("s") * nc + lax.axis_index("c")
        row0 = wid * rows_per_w
        smoothv = jnp.full((L,), smooth, jnp.float32)

        def fill_body(i, carry):
            base = pl.multiple_of(i * (L * FILL_UNROLL), L * FILL_UNROLL)
            for j in range(FILL_UNROLL):
                row_v[pl.ds(base + j * L, L)] = smoothv
            return carry

        lax.fori_loop(0, c // (L * FILL_UNROLL), fill_body, 0)

        fills = []
        for j in range(rows_per_w):
            fills.append(
                pltpu.async_copy(
                    row_v,
                    out_hbm.at[pl.ds((row0 + j) * c, c)],
                    sem_fill,
                )
            )

        pltpu.sync_copy(tgt_hbm.at[pl.ds(row0, rows_per_w)], tgt_v)
        iota = lax.iota(jnp.int32, L)
        for h in range(rows_per_w // L):
            t = tgt_v[pl.ds(h * L, L)]
            rows16 = row0 + h * L + iota
            eidx_v[pl.ds(h * L, L)] = rows16 * c + t
            hot_v[pl.ds(h * L, L)] = jnp.full((L,), hot, jnp.float32)

        for d in fills:
            d.wait()

        pltpu.async_copy(hot_v, out_hbm.at[eidx_v], sem_hot).wait()

    out = sc_kernel(target.astype(jnp.int32))
    return out.reshape(n, c)
